# baseline (speedup 1.0000x reference)
import functools
import jax
import jax.numpy as jnp
from jax import lax
from jax.experimental import pallas as pl
from jax.experimental.pallas import tpu as pltpu
from jax.experimental.pallas import tpu_sc as plsc

N = 10000
E = 320000
D = 128
T = 16
K = 10
ALPHA = 0.1

NC = 2
NS = 16
NW = NC * NS
NP = 10240
EPT = E // NW + 240
CH = 128
NCHUNK = EPT // CH
CH2 = 64
NCHUNK2 = EPT // CH2
ROWS_PT = NP // NS
TRASH = N + 200
BN = 512
GRID = NP // BN

_mesh = plsc.VectorSubcoreMesh(core_axis_name="c", subcore_axis_name="s",
                               num_cores=NC, num_subcores=NS)


def _gelu(x):
  return 0.5 * x * (1.0 + lax.erf(x * 0.7071067811865476))


def _sigmoid(x):
  return 1.0 / (1.0 + jnp.exp(-x))


@functools.partial(
    pl.kernel,
    out_type=jax.ShapeDtypeStruct((NW, EPT), jnp.int32),
    mesh=_mesh,
    scratch_types=[
        pltpu.VMEM((EPT,), jnp.int32),
        pltpu.VMEM((EPT,), jnp.int32),
        pltpu.VMEM((EPT,), jnp.int32),
    ],
)
def _sc_prep(srcf, eff, gidx_out, srcf_v, eff_v, gidx_v):
  c = lax.axis_index("c")
  s = lax.axis_index("s")
  wid = s * NC + c
  pltpu.sync_copy(srcf.at[wid], srcf_v)
  pltpu.sync_copy(eff.at[wid], eff_v)

  def body(k, carry):
    g = eff_v[pl.ds(k * 16, 16)] * NP + srcf_v[pl.ds(k * 16, 16)]
    gidx_v[pl.ds(k * 16, 16)] = g
    return carry

  lax.fori_loop(0, EPT // 16, body, 0)
  pltpu.sync_copy(gidx_v, gidx_out.at[wid])


@functools.partial(
    pl.kernel,
    out_type=jax.ShapeDtypeStruct((NC, NP, D), jnp.float32),
    mesh=_mesh,
    scratch_types=[
        pltpu.VMEM((EPT // 2,), jnp.int32),
        pltpu.VMEM((EPT // 2,), jnp.int32),
        pltpu.VMEM((4, CH2), jnp.int32),
        pltpu.VMEM((4, CH2), jnp.int32),
        pltpu.VMEM((4, CH2, D), jnp.float32),
        pltpu.VMEM_SHARED((NP, D), jnp.float32),
        pltpu.SemaphoreType.DMA,
        pltpu.SemaphoreType.DMA,
        pltpu.SemaphoreType.DMA,
        pltpu.SemaphoreType.DMA,
        pltpu.SemaphoreType.DMA,
    ],
)
def _sc_scatter(fs16, gidxh, dsth, zfeat,
                agg_out,
                gidx_v, didx_v, gib, dib, gbuf, agg,
                gs0, gs1, gs2, gs3, ssem):
  c = lax.axis_index("c")
  s = lax.axis_index("s")
  wid = s * NC + c
  pltpu.sync_copy(zfeat.at[pl.ds(s * ROWS_PT, ROWS_PT)],
                  agg.at[pl.ds(s * ROWS_PT, ROWS_PT)])
  plsc.subcore_barrier()

  gsems = (gs0, gs1, gs2, gs3)

  def stage(j, b):
    for k in range(CH2 // 16):
      gib[b, pl.ds(k * 16, 16)] = gidx_v[pl.ds(j * CH2 + k * 16, 16)]
      dib[b, pl.ds(k * 16, 16)] = didx_v[pl.ds(j * CH2 + k * 16, 16)]

  nch = NCHUNK2 // 2
  for h in range(2):
    pltpu.sync_copy(gidxh.at[wid, h], gidx_v)
    pltpu.sync_copy(dsth.at[wid, h], didx_v)

    def group_body(g, carry):
      j0 = 4 * g
      for b in range(4):
        stage(j0 + b, b)
        pltpu.async_copy(fs16.at[gib.at[b]], gbuf.at[b], gsems[b])
      for b in range(4):
        pltpu.make_async_copy(
            fs16.at[gib.at[b]], gbuf.at[b], gsems[b]).wait()
        pltpu.async_copy(gbuf.at[b], agg.at[dib.at[b]], ssem, add=True)
      for b in range(4):
        pltpu.make_async_copy(gbuf.at[b], agg.at[dib.at[b]], ssem).wait()
      return carry

    lax.fori_loop(0, nch // 4, group_body, 0)

  plsc.subcore_barrier()
  pltpu.sync_copy(agg.at[pl.ds(s * ROWS_PT, ROWS_PT)],
                  agg_out.at[c, pl.ds(s * ROWS_PT, ROWS_PT)])


@functools.partial(
    pl.kernel,
    out_type=jax.ShapeDtypeStruct((NC, NP, D), jnp.float32),
    mesh=_mesh,
    scratch_types=[
        pltpu.VMEM((EPT,), jnp.int32),
        pltpu.VMEM((1, CH2), jnp.int32),
        pltpu.VMEM((CH2, D), jnp.float32),
        pltpu.VMEM_SHARED((NP, D), jnp.float32),
    ],
)
def _sc_deg(idxf, ones, zfeat, agg_out, idx_v, dib, ones_v, agg):
  c = lax.axis_index("c")
  s = lax.axis_index("s")
  wid = s * NC + c
  pltpu.sync_copy(idxf.at[wid], idx_v)
  pltpu.sync_copy(ones, ones_v)
  pltpu.sync_copy(zfeat.at[pl.ds(s * ROWS_PT, ROWS_PT)],
                  agg.at[pl.ds(s * ROWS_PT, ROWS_PT)])
  plsc.subcore_barrier()

  def chunk_body(j, carry):
    for k in range(CH2 // 16):
      dib[0, pl.ds(k * 16, 16)] = idx_v[pl.ds(j * CH2 + k * 16, 16)]
    pltpu.sync_copy(ones_v, agg.at[dib.at[0]], add=True)
    return carry

  lax.fori_loop(0, NCHUNK2, chunk_body, 0)
  plsc.subcore_barrier()
  pltpu.sync_copy(agg.at[pl.ds(s * ROWS_PT, ROWS_PT)],
                  agg_out.at[c, pl.ds(s * ROWS_PT, ROWS_PT)])


def _wtab_body(emb_ref, We1_ref, be1_ref, We2_ref, be2_ref, out_ref):
  h = jnp.dot(emb_ref[...], We1_ref[...],
              preferred_element_type=jnp.float32) + be1_ref[...]
  h = _gelu(h)
  e = jnp.dot(h, We2_ref[...],
              preferred_element_type=jnp.float32) + be2_ref[...]
  out_ref[...] = 1.0 + _sigmoid(e)


_tc_wtab = pl.pallas_call(
    _wtab_body, out_shape=jax.ShapeDtypeStruct((T, 1), jnp.float32))


def _init_body(do_ref, di_ref, f_ref, w_ref, fs16_ref, sn_ref, dn_ref):
  od = do_ref[0, :, 0:1] + do_ref[1, :, 0:1]
  ig = di_ref[0, :, 0:1] + di_ref[1, :, 0:1]
  sn = lax.rsqrt(jnp.maximum(od, 1.0))
  dn = lax.rsqrt(jnp.maximum(ig, 1.0))
  sn_ref[...] = sn
  dn_ref[...] = dn
  fs = sn * f_ref[...]
  fs16_ref[...] = w_ref[...][:, :, None] * fs[None]


_tc_init = pl.pallas_call(
    _init_body,
    grid=(GRID,),
    in_specs=[
        pl.BlockSpec((2, BN, D), lambda i: (0, i, 0)),
        pl.BlockSpec((2, BN, D), lambda i: (0, i, 0)),
        pl.BlockSpec((BN, D), lambda i: (i, 0)),
        pl.BlockSpec((T, 1), lambda i: (0, 0)),
    ],
    out_specs=[
        pl.BlockSpec((T, BN, D), lambda i: (0, i, 0)),
        pl.BlockSpec((BN, 1), lambda i: (i, 0)),
        pl.BlockSpec((BN, 1), lambda i: (i, 0)),
    ],
    out_shape=[
        jax.ShapeDtypeStruct((T, NP, D), jnp.float32),
        jax.ShapeDtypeStruct((NP, 1), jnp.float32),
        jax.ShapeDtypeStruct((NP, 1), jnp.float32),
    ],
)


def _update_body(agg_ref, f0_ref, dn_ref, sn_ref, w_ref, fs16_ref):
  a = agg_ref[0] + agg_ref[1]
  f = (1.0 - ALPHA) * (dn_ref[...] * a) + ALPHA * f0_ref[...]
  fs = sn_ref[...] * f
  fs16_ref[...] = w_ref[...][:, :, None] * fs[None]


_tc_update = pl.pallas_call(
    _update_body,
    grid=(GRID,),
    in_specs=[
        pl.BlockSpec((2, BN, D), lambda i: (0, i, 0)),
        pl.BlockSpec((BN, D), lambda i: (i, 0)),
        pl.BlockSpec((BN, 1), lambda i: (i, 0)),
        pl.BlockSpec((BN, 1), lambda i: (i, 0)),
        pl.BlockSpec((T, 1), lambda i: (0, 0)),
    ],
    out_specs=pl.BlockSpec((T, BN, D), lambda i: (0, i, 0)),
    out_shape=jax.ShapeDtypeStruct((T, NP, D), jnp.float32),
)


def _final_body(agg_ref, f0_ref, dn_ref, W1_ref, b1_ref, W2_ref, b2_ref,
                out_ref):
  a = agg_ref[0] + agg_ref[1]
  f = (1.0 - ALPHA) * (dn_ref[...] * a) + ALPHA * f0_ref[...]
  h = jnp.dot(f, W1_ref[...], preferred_element_type=jnp.float32) + b1_ref[...]
  h = _gelu(h)
  out_ref[...] = jnp.dot(
      h, W2_ref[...], preferred_element_type=jnp.float32) + b2_ref[...]


_tc_final = pl.pallas_call(
    _final_body,
    grid=(GRID,),
    in_specs=[
        pl.BlockSpec((2, BN, D), lambda i: (0, i, 0)),
        pl.BlockSpec((BN, D), lambda i: (i, 0)),
        pl.BlockSpec((BN, 1), lambda i: (i, 0)),
        pl.BlockSpec((D, D), lambda i: (0, 0)),
        pl.BlockSpec((1, D), lambda i: (0, 0)),
        pl.BlockSpec((D, D), lambda i: (0, 0)),
        pl.BlockSpec((1, D), lambda i: (0, 0)),
    ],
    out_specs=pl.BlockSpec((BN, D), lambda i: (i, 0)),
    out_shape=jax.ShapeDtypeStruct((NP, D), jnp.float32),
)


def kernel(feat, edge_index, e_feat, embed_table, We1, be1, We2, be2,
           W1, b1, W2, b2):
  src = edge_index[0]
  dst = edge_index[1]
  pad_w = ((0, 0), (0, EPT - E // NW))
  srcp = jnp.pad(src.reshape(NW, E // NW), pad_w, constant_values=TRASH)
  dstp = jnp.pad(dst.reshape(NW, E // NW), pad_w, constant_values=TRASH)
  efp = jnp.pad(e_feat.reshape(NW, E // NW), pad_w, constant_values=0)
  dst2b = dstp.reshape(NW, 2, EPT // 2)
  onesrows = jnp.ones((CH2, D), jnp.float32)
  zfeat = jnp.zeros((NP, D), jnp.float32)
  featp = jnp.pad(feat, ((0, NP - N), (0, 0)))
  be1r = be1.reshape(1, 32)
  be2r = be2.reshape(1, 1)
  b1r = b1.reshape(1, D)
  b2r = b2.reshape(1, D)

  wtab = _tc_wtab(embed_table, We1, be1r, We2, be2r)
  gidxh = _sc_prep(srcp, efp)
  gidxh2 = gidxh.reshape(NW, 2, EPT // 2)
  agg_od = _sc_deg(srcp, onesrows, zfeat)
  agg_id = _sc_deg(dstp, onesrows, zfeat)
  fs16, sn, dn = _tc_init(agg_od, agg_id, featp, wtab)
  agg = None
  for it in range(K):
    agg = _sc_scatter(fs16.reshape(T * NP, D), gidxh2, dst2b, zfeat)
    if it < K - 1:
      fs16 = _tc_update(agg, featp, dn, sn, wtab)
  out = _tc_final(agg, featp, dn, W1, b1r, W2, b2r)
  return out[:N]


# speedup vs baseline: 1.0301x; 1.0301x over previous
import functools
import jax
import jax.numpy as jnp
from jax import lax
from jax.experimental import pallas as pl
from jax.experimental.pallas import tpu as pltpu
from jax.experimental.pallas import tpu_sc as plsc

N = 10000
E = 320000
D = 128
T = 16
K = 10
ALPHA = 0.1

NC = 2
NS = 16
NW = NC * NS
NP = 10240
EPT = E // NW + 240
CH = 128
NCHUNK = EPT // CH
CH2 = 64
NCHUNK2 = EPT // CH2
ROWS_PT = NP // NS
TRASH = N + 200
BN = 512
GRID = NP // BN

_mesh = plsc.VectorSubcoreMesh(core_axis_name="c", subcore_axis_name="s",
                               num_cores=NC, num_subcores=NS)


def _gelu(x):
  return 0.5 * x * (1.0 + lax.erf(x * 0.7071067811865476))


def _sigmoid(x):
  return 1.0 / (1.0 + jnp.exp(-x))


@functools.partial(
    pl.kernel,
    out_type=jax.ShapeDtypeStruct((NW, EPT), jnp.int32),
    mesh=_mesh,
    scratch_types=[
        pltpu.VMEM((EPT,), jnp.int32),
        pltpu.VMEM((EPT,), jnp.int32),
        pltpu.VMEM((EPT,), jnp.int32),
    ],
)
def _sc_prep(srcf, eff, gidx_out, srcf_v, eff_v, gidx_v):
  c = lax.axis_index("c")
  s = lax.axis_index("s")
  wid = s * NC + c
  pltpu.sync_copy(srcf.at[wid], srcf_v)
  pltpu.sync_copy(eff.at[wid], eff_v)

  def body(k, carry):
    g = eff_v[pl.ds(k * 16, 16)] * NP + srcf_v[pl.ds(k * 16, 16)]
    gidx_v[pl.ds(k * 16, 16)] = g
    return carry

  lax.fori_loop(0, EPT // 16, body, 0)
  pltpu.sync_copy(gidx_v, gidx_out.at[wid])


@functools.partial(
    pl.kernel,
    out_type=jax.ShapeDtypeStruct((NC, NP, D), jnp.float32),
    mesh=_mesh,
    scratch_types=[
        pltpu.VMEM((EPT // 2,), jnp.int32),
        pltpu.VMEM((EPT // 2,), jnp.int32),
        pltpu.VMEM((2, CH2), jnp.int32),
        pltpu.VMEM((2, CH2), jnp.int32),
        pltpu.VMEM((2, CH2, D), jnp.float32),
        pltpu.VMEM_SHARED((NP, D), jnp.float32),
        pltpu.SemaphoreType.DMA,
        pltpu.SemaphoreType.DMA,
    ],
)
def _sc_scatter(fs16, gidxh, dsth, zfeat,
                agg_out,
                gidx_v, didx_v, gib, dib, gbuf, agg, sem0, sem1):
  c = lax.axis_index("c")
  s = lax.axis_index("s")
  wid = s * NC + c
  pltpu.sync_copy(zfeat.at[pl.ds(s * ROWS_PT, ROWS_PT)],
                  agg.at[pl.ds(s * ROWS_PT, ROWS_PT)])
  plsc.subcore_barrier()

  sems = (sem0, sem1)

  def stage(j, b):
    for k in range(CH2 // 16):
      gib[b, pl.ds(k * 16, 16)] = gidx_v[pl.ds(j * CH2 + k * 16, 16)]
      dib[b, pl.ds(k * 16, 16)] = didx_v[pl.ds(j * CH2 + k * 16, 16)]

  def gath(b):
    pltpu.async_copy(fs16.at[gib.at[b]], gbuf.at[b], sems[b])

  def scat(b):
    pltpu.make_async_copy(fs16.at[gib.at[b]], gbuf.at[b], sems[b]).wait()
    pltpu.sync_copy(gbuf.at[b], agg.at[dib.at[b]], add=True)

  nch = NCHUNK2 // 2
  for h in range(2):
    pltpu.sync_copy(gidxh.at[wid, h], gidx_v)
    pltpu.sync_copy(dsth.at[wid, h], didx_v)
    stage(0, 0)
    gath(0)

    def chunk_body(g, carry):
      j0 = 2 * g
      stage(j0 + 1, 1)
      gath(1)
      scat(0)
      stage(j0 + 2, 0)
      gath(0)
      scat(1)
      return carry

    lax.fori_loop(0, nch // 2 - 1, chunk_body, 0)
    stage(nch - 1, 1)
    gath(1)
    scat(0)
    scat(1)

  plsc.subcore_barrier()
  pltpu.sync_copy(agg.at[pl.ds(s * ROWS_PT, ROWS_PT)],
                  agg_out.at[c, pl.ds(s * ROWS_PT, ROWS_PT)])


@functools.partial(
    pl.kernel,
    out_type=jax.ShapeDtypeStruct((NC, NP, D), jnp.float32),
    mesh=_mesh,
    scratch_types=[
        pltpu.VMEM((EPT,), jnp.int32),
        pltpu.VMEM((1, CH2), jnp.int32),
        pltpu.VMEM((CH2, D), jnp.float32),
        pltpu.VMEM_SHARED((NP, D), jnp.float32),
    ],
)
def _sc_deg(idxf, ones, zfeat, agg_out, idx_v, dib, ones_v, agg):
  c = lax.axis_index("c")
  s = lax.axis_index("s")
  wid = s * NC + c
  pltpu.sync_copy(idxf.at[wid], idx_v)
  pltpu.sync_copy(ones, ones_v)
  pltpu.sync_copy(zfeat.at[pl.ds(s * ROWS_PT, ROWS_PT)],
                  agg.at[pl.ds(s * ROWS_PT, ROWS_PT)])
  plsc.subcore_barrier()

  def chunk_body(j, carry):
    for k in range(CH2 // 16):
      dib[0, pl.ds(k * 16, 16)] = idx_v[pl.ds(j * CH2 + k * 16, 16)]
    pltpu.sync_copy(ones_v, agg.at[dib.at[0]], add=True)
    return carry

  lax.fori_loop(0, NCHUNK2, chunk_body, 0)
  plsc.subcore_barrier()
  pltpu.sync_copy(agg.at[pl.ds(s * ROWS_PT, ROWS_PT)],
                  agg_out.at[c, pl.ds(s * ROWS_PT, ROWS_PT)])


def _wtab_body(emb_ref, We1_ref, be1_ref, We2_ref, be2_ref, out_ref):
  h = jnp.dot(emb_ref[...], We1_ref[...],
              preferred_element_type=jnp.float32) + be1_ref[...]
  h = _gelu(h)
  e = jnp.dot(h, We2_ref[...],
              preferred_element_type=jnp.float32) + be2_ref[...]
  out_ref[...] = 1.0 + _sigmoid(e)


_tc_wtab = pl.pallas_call(
    _wtab_body, out_shape=jax.ShapeDtypeStruct((T, 1), jnp.float32))


def _init_body(do_ref, di_ref, f_ref, w_ref, fs16_ref, sn_ref, dn_ref):
  od = do_ref[0, :, 0:1] + do_ref[1, :, 0:1]
  ig = di_ref[0, :, 0:1] + di_ref[1, :, 0:1]
  sn = lax.rsqrt(jnp.maximum(od, 1.0))
  dn = lax.rsqrt(jnp.maximum(ig, 1.0))
  sn_ref[...] = sn
  dn_ref[...] = dn
  fs = sn * f_ref[...]
  fs16_ref[...] = w_ref[...][:, :, None] * fs[None]


_tc_init = pl.pallas_call(
    _init_body,
    grid=(GRID,),
    in_specs=[
        pl.BlockSpec((2, BN, D), lambda i: (0, i, 0)),
        pl.BlockSpec((2, BN, D), lambda i: (0, i, 0)),
        pl.BlockSpec((BN, D), lambda i: (i, 0)),
        pl.BlockSpec((T, 1), lambda i: (0, 0)),
    ],
    out_specs=[
        pl.BlockSpec((T, BN, D), lambda i: (0, i, 0)),
        pl.BlockSpec((BN, 1), lambda i: (i, 0)),
        pl.BlockSpec((BN, 1), lambda i: (i, 0)),
    ],
    out_shape=[
        jax.ShapeDtypeStruct((T, NP, D), jnp.float32),
        jax.ShapeDtypeStruct((NP, 1), jnp.float32),
        jax.ShapeDtypeStruct((NP, 1), jnp.float32),
    ],
)


def _update_body(agg_ref, f0_ref, dn_ref, sn_ref, w_ref, fs16_ref):
  a = agg_ref[0] + agg_ref[1]
  f = (1.0 - ALPHA) * (dn_ref[...] * a) + ALPHA * f0_ref[...]
  fs = sn_ref[...] * f
  fs16_ref[...] = w_ref[...][:, :, None] * fs[None]


_tc_update = pl.pallas_call(
    _update_body,
    grid=(GRID,),
    in_specs=[
        pl.BlockSpec((2, BN, D), lambda i: (0, i, 0)),
        pl.BlockSpec((BN, D), lambda i: (i, 0)),
        pl.BlockSpec((BN, 1), lambda i: (i, 0)),
        pl.BlockSpec((BN, 1), lambda i: (i, 0)),
        pl.BlockSpec((T, 1), lambda i: (0, 0)),
    ],
    out_specs=pl.BlockSpec((T, BN, D), lambda i: (0, i, 0)),
    out_shape=jax.ShapeDtypeStruct((T, NP, D), jnp.float32),
)


def _final_body(agg_ref, f0_ref, dn_ref, W1_ref, b1_ref, W2_ref, b2_ref,
                out_ref):
  a = agg_ref[0] + agg_ref[1]
  f = (1.0 - ALPHA) * (dn_ref[...] * a) + ALPHA * f0_ref[...]
  h = jnp.dot(f, W1_ref[...], preferred_element_type=jnp.float32) + b1_ref[...]
  h = _gelu(h)
  out_ref[...] = jnp.dot(
      h, W2_ref[...], preferred_element_type=jnp.float32) + b2_ref[...]


_tc_final = pl.pallas_call(
    _final_body,
    grid=(GRID,),
    in_specs=[
        pl.BlockSpec((2, BN, D), lambda i: (0, i, 0)),
        pl.BlockSpec((BN, D), lambda i: (i, 0)),
        pl.BlockSpec((BN, 1), lambda i: (i, 0)),
        pl.BlockSpec((D, D), lambda i: (0, 0)),
        pl.BlockSpec((1, D), lambda i: (0, 0)),
        pl.BlockSpec((D, D), lambda i: (0, 0)),
        pl.BlockSpec((1, D), lambda i: (0, 0)),
    ],
    out_specs=pl.BlockSpec((BN, D), lambda i: (i, 0)),
    out_shape=jax.ShapeDtypeStruct((NP, D), jnp.float32),
)


def kernel(feat, edge_index, e_feat, embed_table, We1, be1, We2, be2,
           W1, b1, W2, b2):
  src = edge_index[0]
  dst = edge_index[1]
  pad_w = ((0, 0), (0, EPT - E // NW))
  srcp = jnp.pad(src.reshape(NW, E // NW), pad_w, constant_values=TRASH)
  dstp = jnp.pad(dst.reshape(NW, E // NW), pad_w, constant_values=TRASH)
  efp = jnp.pad(e_feat.reshape(NW, E // NW), pad_w, constant_values=0)
  dst2b = dstp.reshape(NW, 2, EPT // 2)
  onesrows = jnp.ones((CH2, D), jnp.float32)
  zfeat = jnp.zeros((NP, D), jnp.float32)
  featp = jnp.pad(feat, ((0, NP - N), (0, 0)))
  be1r = be1.reshape(1, 32)
  be2r = be2.reshape(1, 1)
  b1r = b1.reshape(1, D)
  b2r = b2.reshape(1, D)

  wtab = _tc_wtab(embed_table, We1, be1r, We2, be2r)
  gidxh = _sc_prep(srcp, efp)
  gidxh2 = gidxh.reshape(NW, 2, EPT // 2)
  agg_od = _sc_deg(srcp, onesrows, zfeat)
  agg_id = _sc_deg(dstp, onesrows, zfeat)
  fs16, sn, dn = _tc_init(agg_od, agg_id, featp, wtab)
  agg = None
  for it in range(K):
    agg = _sc_scatter(fs16.reshape(T * NP, D), gidxh2, dst2b, zfeat)
    if it < K - 1:
      fs16 = _tc_update(agg, featp, dn, sn, wtab)
  out = _tc_final(agg, featp, dn, W1, b1r, W2, b2r)
  return out[:N]
